# baseline (speedup 1.0000x reference)
import functools

import jax
import jax.numpy as jnp
import numpy as np
from jax import lax
from jax.experimental import pallas as pl
from jax.experimental.pallas import tpu as pltpu
from jax.experimental.pallas import tpu_sc as plsc

B = 16384
NUM = 13
F = 26
V = 100000
D = 32
NF = F + 1
NPAIR = NF * (NF - 1) // 2
NPAD = 352

_NC, _NS = 2, 16
NW = _NC * _NS
CH = 128
NG = B // CH
NCHT = F * NG
NCH = NCHT // NW


@functools.lru_cache(maxsize=None)
def _make_sc_gather():
    mesh = plsc.VectorSubcoreMesh(core_axis_name="c", subcore_axis_name="s")
    return functools.partial(
        pl.kernel,
        out_type=jax.ShapeDtypeStruct((NG * F * D * CH,), jnp.float32),
        mesh=mesh,
        scratch_types=[
            pltpu.VMEM((NCH, CH), jnp.int32),
            pltpu.VMEM((CH, D), jnp.float32),
            pltpu.VMEM((CH, D), jnp.float32),
            pltpu.VMEM((D * CH,), jnp.float32),
            pltpu.SemaphoreType.DMA,
            pltpu.SemaphoreType.DMA,
        ],
        compiler_params=pltpu.CompilerParams(
            use_tc_tiling_on_sc=False, needs_layout_passes=False),
    )(_sc_gather_body)


def _transpose_chunk(src, dst):
    for h in range(CH // 16):
        rows = h * 16 + lax.iota(jnp.int32, 16)
        for d in range(D):
            cols = jnp.full((16,), d, jnp.int32)
            vals = plsc.load_gather(src, [rows, cols])
            dst[pl.ds(d * CH + h * 16, 16)] = vals


def _sc_gather_body(table_hbm, idx_hbm, out_hbm, idx_v, buf0, buf1, bufT,
                    sem0, sem1):
    wid = lax.axis_index("s") * _NC + lax.axis_index("c")
    cbase = wid * NCH
    pltpu.sync_copy(idx_hbm.at[wid], idx_v)
    pltpu.async_copy(table_hbm.at[idx_v.at[0]], buf0, sem0)

    def flush(j, buf, sem):
        c = cbase + j
        f = c // NG
        g = c - f * NG
        pltpu.make_async_copy(table_hbm.at[idx_v.at[j]], buf, sem).wait()
        _transpose_chunk(buf, bufT)
        pltpu.sync_copy(bufT, out_hbm.at[pl.ds((g * F * D + f * D) * CH, D * CH)])

    def body(k, carry):
        j0 = 2 * k
        pltpu.async_copy(table_hbm.at[idx_v.at[j0 + 1]], buf1, sem1)
        flush(j0, buf0, sem0)

        @pl.when(j0 + 2 < NCH)
        def _():
            pltpu.async_copy(table_hbm.at[idx_v.at[j0 + 2]], buf0, sem0)

        flush(j0 + 1, buf1, sem1)
        return carry

    lax.fori_loop(0, NCH // 2, body, 0)


BK = 512
GPB = BK // CH


def _tc_body(num_ref, embs_ref,
             bw0_ref, bb0_ref, bw1_ref, bb1_ref, bw2_ref, bb2_ref,
             tw0a_ref, tw0b_ref, tb0_ref, tw1_ref, tb1_ref, tw2_ref, tb2_ref,
             tw3_ref, tb3_ref, tw4_ref, tb4_ref, out_ref, zscr):
    nm = num_ref[...]
    h = jnp.maximum(bw0_ref[...] @ nm + bb0_ref[...], 0.0)
    h = jnp.maximum(bw1_ref[...] @ h + bb1_ref[...], 0.0)
    bot = jnp.maximum(bw2_ref[...] @ h + bb2_ref[...], 0.0)
    em = embs_ref[...].reshape(GPB * F * D, CH)
    et = jnp.concatenate(
        [em[i * (F * D):(i + 1) * (F * D), :] for i in range(GPB)], axis=1)
    tfull = jnp.concatenate([bot, et], axis=0)
    tfull = tfull.astype(jnp.bfloat16).astype(jnp.float32)
    for n in range(1, NF):
        vn = tfull[n * D:(n + 1) * D, :]
        pre = tfull[0:n * D, :].reshape(n, D, BK)
        pr = jnp.sum(pre * vn[None, :, :], axis=1)
        zscr[pl.ds(n * (n - 1) // 2, n), :] = pr
    zscr[NPAIR] = jnp.zeros((BK,), jnp.float32)
    zt = zscr[...]
    h = jnp.maximum(tw0a_ref[...] @ bot + tw0b_ref[...] @ zt + tb0_ref[...], 0.0)
    h = jnp.maximum(tw1_ref[...] @ h + tb1_ref[...], 0.0)
    h = jnp.maximum(tw2_ref[...] @ h + tb2_ref[...], 0.0)
    h = jnp.maximum(tw3_ref[...] @ h + tb3_ref[...], 0.0)
    out_ref[...] = tw4_ref[...] @ h + tb4_ref[...]


def _full2(shape):
    return pl.BlockSpec(shape, lambda i: (0, 0))


def _tc_fused(numT, embs, bw0t, bb0c, bw1t, bb1c, bw2t, bb2c,
              tw0at, tw0bt, tb0c, tw1t, tb1c, tw2t, tb2c, tw3t, tb3c,
              tw4t, tb4c):
    grid = (B // BK,)
    in_specs = [
        pl.BlockSpec((NUM, BK), lambda i: (0, i)),
        pl.BlockSpec((GPB * F * D * CH,), lambda i: (i,)),
        _full2(bw0t.shape), _full2(bb0c.shape), _full2(bw1t.shape),
        _full2(bb1c.shape), _full2(bw2t.shape), _full2(bb2c.shape),
        _full2(tw0at.shape), _full2(tw0bt.shape), _full2(tb0c.shape),
        _full2(tw1t.shape), _full2(tb1c.shape), _full2(tw2t.shape),
        _full2(tb2c.shape), _full2(tw3t.shape), _full2(tb3c.shape),
        _full2(tw4t.shape), _full2(tb4c.shape),
    ]
    return pl.pallas_call(
        _tc_body,
        grid=grid,
        in_specs=in_specs,
        out_specs=pl.BlockSpec((1, BK), lambda i: (0, i)),
        out_shape=jax.ShapeDtypeStruct((1, B), jnp.float32),
        scratch_shapes=[pltpu.VMEM((NPAD, BK), jnp.float32)],
    )(numT, embs, bw0t, bb0c, bw1t, bb1c, bw2t, bb2c,
      tw0at, tw0bt, tb0c, tw1t, tb1c, tw2t, tb2c, tw3t, tb3c, tw4t, tb4c)


def kernel(numerical_input, categorical_inputs, emb_tables,
           bw0, bb0, bw1, bb1, bw2, bb2,
           tw0, tb0, tw1, tb1, tw2, tb2, tw3, tb3, tw4, tb4):
    cat = categorical_inputs.astype(jnp.int32)
    idxT = cat.T + (jnp.arange(F, dtype=jnp.int32) * V)[:, None]
    idx3 = idxT.reshape(NCHT, CH).reshape(NW, NCH, CH)
    table = emb_tables.reshape(F * V, D)
    embs = _make_sc_gather()(table, idx3)

    outT = _tc_fused(
        numerical_input.T, embs,
        bw0.T, bb0.reshape(-1, 1), bw1.T, bb1.reshape(-1, 1),
        bw2.T, bb2.reshape(-1, 1),
        tw0[:D].T, jnp.pad(tw0[D:D + NPAIR], ((0, 1), (0, 0))).T,
        tb0.reshape(-1, 1), tw1.T, tb1.reshape(-1, 1), tw2.T,
        tb2.reshape(-1, 1), tw3.T, tb3.reshape(-1, 1), tw4.T,
        tb4.reshape(-1, 1))
    return outT.T


# speedup vs baseline: 1.0119x; 1.0119x over previous
import functools

import jax
import jax.numpy as jnp
import numpy as np
from jax import lax
from jax.experimental import pallas as pl
from jax.experimental.pallas import tpu as pltpu
from jax.experimental.pallas import tpu_sc as plsc

B = 16384
NUM = 13
F = 26
V = 100000
D = 32
NF = F + 1
NPAIR = NF * (NF - 1) // 2
NPAD = 352

_NC, _NS = 2, 16
NW = _NC * _NS
ROWS = B * F
CH = 128
NCH = ROWS // NW // CH


@functools.lru_cache(maxsize=None)
def _make_sc_gather():
    mesh = plsc.VectorSubcoreMesh(core_axis_name="c", subcore_axis_name="s")
    return functools.partial(
        pl.kernel,
        out_type=jax.ShapeDtypeStruct((ROWS, D), jnp.float32),
        mesh=mesh,
        scratch_types=[
            pltpu.VMEM((NCH, CH), jnp.int32),
            pltpu.VMEM((CH, D), jnp.float32),
            pltpu.VMEM((CH, D), jnp.float32),
            pltpu.SemaphoreType.DMA,
            pltpu.SemaphoreType.DMA,
        ],
        compiler_params=pltpu.CompilerParams(
            use_tc_tiling_on_sc=False, needs_layout_passes=False),
    )(_sc_gather_body)


def _sc_gather_body(table_hbm, idx_hbm, out_hbm, idx_v, buf0, buf1,
                    sem0, sem1):
    wid = lax.axis_index("s") * _NC + lax.axis_index("c")
    cbase = wid * NCH
    pltpu.sync_copy(idx_hbm.at[wid], idx_v)
    pltpu.async_copy(table_hbm.at[idx_v.at[0]], buf0, sem0)

    def flush(j, buf, sem):
        pltpu.make_async_copy(table_hbm.at[idx_v.at[j]], buf, sem).wait()
        pltpu.sync_copy(buf, out_hbm.at[pl.ds((cbase + j) * CH, CH)])

    def body(k, carry):
        j0 = 2 * k
        pltpu.async_copy(table_hbm.at[idx_v.at[j0 + 1]], buf1, sem1)
        flush(j0, buf0, sem0)

        @pl.when(j0 + 2 < NCH)
        def _():
            pltpu.async_copy(table_hbm.at[idx_v.at[j0 + 2]], buf0, sem0)

        flush(j0 + 1, buf1, sem1)
        return carry

    lax.fori_loop(0, NCH // 2, body, 0)


BK = 512
GPB = BK // CH


def _tc_body(num_ref, embs_ref,
             bw0_ref, bb0_ref, bw1_ref, bb1_ref, bw2_ref, bb2_ref,
             tw0a_ref, tw0b_ref, tb0_ref, tw1_ref, tb1_ref, tw2_ref, tb2_ref,
             tw3_ref, tb3_ref, tw4_ref, tb4_ref, out_ref, zscr):
    nm = num_ref[...]
    h = jnp.maximum(bw0_ref[...] @ nm + bb0_ref[...], 0.0)
    h = jnp.maximum(bw1_ref[...] @ h + bb1_ref[...], 0.0)
    bot = jnp.maximum(bw2_ref[...] @ h + bb2_ref[...], 0.0)
    em = embs_ref[...]
    et = jnp.transpose(em, (1, 2, 0)).reshape(F * D, BK)
    tfull = jnp.concatenate([bot, et], axis=0)
    tfull = tfull.astype(jnp.bfloat16).astype(jnp.float32)
    for n in range(1, NF):
        vn = tfull[n * D:(n + 1) * D, :]
        pre = tfull[0:n * D, :].reshape(n, D, BK)
        pr = jnp.sum(pre * vn[None, :, :], axis=1)
        zscr[pl.ds(n * (n - 1) // 2, n), :] = pr
    zscr[NPAIR] = jnp.zeros((BK,), jnp.float32)
    zt = zscr[...]
    h = jnp.maximum(tw0a_ref[...] @ bot + tw0b_ref[...] @ zt + tb0_ref[...], 0.0)
    h = jnp.maximum(tw1_ref[...] @ h + tb1_ref[...], 0.0)
    h = jnp.maximum(tw2_ref[...] @ h + tb2_ref[...], 0.0)
    h = jnp.maximum(tw3_ref[...] @ h + tb3_ref[...], 0.0)
    out_ref[...] = tw4_ref[...] @ h + tb4_ref[...]


def _full2(shape):
    return pl.BlockSpec(shape, lambda i: (0, 0))


def _tc_fused(numT, embs, bw0t, bb0c, bw1t, bb1c, bw2t, bb2c,
              tw0at, tw0bt, tb0c, tw1t, tb1c, tw2t, tb2c, tw3t, tb3c,
              tw4t, tb4c):
    grid = (B // BK,)
    in_specs = [
        pl.BlockSpec((NUM, BK), lambda i: (0, i)),
        pl.BlockSpec((BK, F, D), lambda i: (i, 0, 0)),
        _full2(bw0t.shape), _full2(bb0c.shape), _full2(bw1t.shape),
        _full2(bb1c.shape), _full2(bw2t.shape), _full2(bb2c.shape),
        _full2(tw0at.shape), _full2(tw0bt.shape), _full2(tb0c.shape),
        _full2(tw1t.shape), _full2(tb1c.shape), _full2(tw2t.shape),
        _full2(tb2c.shape), _full2(tw3t.shape), _full2(tb3c.shape),
        _full2(tw4t.shape), _full2(tb4c.shape),
    ]
    return pl.pallas_call(
        _tc_body,
        grid=grid,
        in_specs=in_specs,
        out_specs=pl.BlockSpec((1, BK), lambda i: (0, i)),
        out_shape=jax.ShapeDtypeStruct((1, B), jnp.float32),
        scratch_shapes=[pltpu.VMEM((NPAD, BK), jnp.float32)],
    )(numT, embs, bw0t, bb0c, bw1t, bb1c, bw2t, bb2c,
      tw0at, tw0bt, tb0c, tw1t, tb1c, tw2t, tb2c, tw3t, tb3c, tw4t, tb4c)


def kernel(numerical_input, categorical_inputs, emb_tables,
           bw0, bb0, bw1, bb1, bw2, bb2,
           tw0, tb0, tw1, tb1, tw2, tb2, tw3, tb3, tw4, tb4):
    cat = categorical_inputs.astype(jnp.int32)
    flat_idx = (cat + (jnp.arange(F, dtype=jnp.int32) * V)[None, :]).reshape(-1)
    idx3 = flat_idx.reshape(NW, NCH, CH)
    table = emb_tables.reshape(F * V, D)
    embs = _make_sc_gather()(table, idx3).reshape(B, F, D)

    outT = _tc_fused(
        numerical_input.T, embs,
        bw0.T, bb0.reshape(-1, 1), bw1.T, bb1.reshape(-1, 1),
        bw2.T, bb2.reshape(-1, 1),
        tw0[:D].T, jnp.pad(tw0[D:D + NPAIR], ((0, 1), (0, 0))).T,
        tb0.reshape(-1, 1), tw1.T, tb1.reshape(-1, 1), tw2.T,
        tb2.reshape(-1, 1), tw3.T, tb3.reshape(-1, 1), tw4.T,
        tb4.reshape(-1, 1))
    return outT.T


# speedup vs baseline: 1.1707x; 1.1569x over previous
import functools

import jax
import jax.numpy as jnp
import numpy as np
from jax import lax
from jax.experimental import pallas as pl
from jax.experimental.pallas import tpu as pltpu
from jax.experimental.pallas import tpu_sc as plsc

B = 16384
NUM = 13
F = 26
V = 100000
D = 32
NF = F + 1
NPAIR = NF * (NF - 1) // 2
NPAD = 352

_NC, _NS = 2, 16
NW = _NC * _NS
ROWS = B * F
CH = 128
NCH = ROWS // NW // CH


@functools.lru_cache(maxsize=None)
def _make_sc_gather():
    mesh = plsc.VectorSubcoreMesh(core_axis_name="c", subcore_axis_name="s")
    return functools.partial(
        pl.kernel,
        out_type=jax.ShapeDtypeStruct((ROWS, D), jnp.float32),
        mesh=mesh,
        scratch_types=[
            pltpu.VMEM((NCH, CH), jnp.int32),
            pltpu.VMEM((CH, D), jnp.float32),
            pltpu.VMEM((CH, D), jnp.float32),
            pltpu.SemaphoreType.DMA,
            pltpu.SemaphoreType.DMA,
        ],
        compiler_params=pltpu.CompilerParams(
            use_tc_tiling_on_sc=False, needs_layout_passes=False),
    )(_sc_gather_body)


def _sc_gather_body(table_hbm, idx_hbm, out_hbm, idx_v, buf0, buf1,
                    sem0, sem1):
    wid = lax.axis_index("s") * _NC + lax.axis_index("c")
    cbase = wid * NCH
    pltpu.sync_copy(idx_hbm.at[wid], idx_v)
    pltpu.async_copy(table_hbm.at[idx_v.at[0]], buf0, sem0)

    def flush(j, buf, sem):
        pltpu.make_async_copy(table_hbm.at[idx_v.at[j]], buf, sem).wait()
        pltpu.sync_copy(buf, out_hbm.at[pl.ds((cbase + j) * CH, CH)])

    def body(k, carry):
        j0 = 2 * k
        pltpu.async_copy(table_hbm.at[idx_v.at[j0 + 1]], buf1, sem1)
        flush(j0, buf0, sem0)

        @pl.when(j0 + 2 < NCH)
        def _():
            pltpu.async_copy(table_hbm.at[idx_v.at[j0 + 2]], buf0, sem0)

        flush(j0 + 1, buf1, sem1)
        return carry

    lax.fori_loop(0, NCH // 2, body, 0)


BK = 512
GPB = BK // CH


def _tc_body(num_ref, embs_ref,
             bw0_ref, bb0_ref, bw1_ref, bb1_ref, bw2_ref, bb2_ref,
             tw0a_ref, tw0b_ref, tb0_ref, tw1_ref, tb1_ref, tw2_ref, tb2_ref,
             tw3_ref, tb3_ref, tw4_ref, tb4_ref, out_ref, zscr):
    nm = num_ref[...]
    h = jnp.maximum(bw0_ref[...] @ nm + bb0_ref[...], 0.0)
    h = jnp.maximum(bw1_ref[...] @ h + bb1_ref[...], 0.0)
    bot = jnp.maximum(bw2_ref[...] @ h + bb2_ref[...], 0.0)
    em = embs_ref[...]
    et = em.T
    tfull = jnp.concatenate([bot, et], axis=0)
    tfull = tfull.astype(jnp.bfloat16).astype(jnp.float32)
    for n in range(1, NF):
        vn = tfull[n * D:(n + 1) * D, :]
        pre = tfull[0:n * D, :].reshape(n, D, BK)
        pr = jnp.sum(pre * vn[None, :, :], axis=1)
        zscr[pl.ds(n * (n - 1) // 2, n), :] = pr
    zscr[NPAIR] = jnp.zeros((BK,), jnp.float32)
    zt = zscr[...]
    h = jnp.maximum(tw0a_ref[...] @ bot + tw0b_ref[...] @ zt + tb0_ref[...], 0.0)
    h = jnp.maximum(tw1_ref[...] @ h + tb1_ref[...], 0.0)
    h = jnp.maximum(tw2_ref[...] @ h + tb2_ref[...], 0.0)
    h = jnp.maximum(tw3_ref[...] @ h + tb3_ref[...], 0.0)
    out_ref[...] = tw4_ref[...] @ h + tb4_ref[...]


def _full2(shape):
    return pl.BlockSpec(shape, lambda i: (0, 0))


def _tc_fused(numT, embs, bw0t, bb0c, bw1t, bb1c, bw2t, bb2c,
              tw0at, tw0bt, tb0c, tw1t, tb1c, tw2t, tb2c, tw3t, tb3c,
              tw4t, tb4c):
    grid = (B // BK,)
    in_specs = [
        pl.BlockSpec((NUM, BK), lambda i: (0, i)),
        pl.BlockSpec((BK, F * D), lambda i: (i, 0)),
        _full2(bw0t.shape), _full2(bb0c.shape), _full2(bw1t.shape),
        _full2(bb1c.shape), _full2(bw2t.shape), _full2(bb2c.shape),
        _full2(tw0at.shape), _full2(tw0bt.shape), _full2(tb0c.shape),
        _full2(tw1t.shape), _full2(tb1c.shape), _full2(tw2t.shape),
        _full2(tb2c.shape), _full2(tw3t.shape), _full2(tb3c.shape),
        _full2(tw4t.shape), _full2(tb4c.shape),
    ]
    return pl.pallas_call(
        _tc_body,
        grid=grid,
        in_specs=in_specs,
        out_specs=pl.BlockSpec((1, BK), lambda i: (0, i)),
        out_shape=jax.ShapeDtypeStruct((1, B), jnp.float32),
        scratch_shapes=[pltpu.VMEM((NPAD, BK), jnp.float32)],
    )(numT, embs, bw0t, bb0c, bw1t, bb1c, bw2t, bb2c,
      tw0at, tw0bt, tb0c, tw1t, tb1c, tw2t, tb2c, tw3t, tb3c, tw4t, tb4c)


def kernel(numerical_input, categorical_inputs, emb_tables,
           bw0, bb0, bw1, bb1, bw2, bb2,
           tw0, tb0, tw1, tb1, tw2, tb2, tw3, tb3, tw4, tb4):
    cat = categorical_inputs.astype(jnp.int32)
    flat_idx = (cat + (jnp.arange(F, dtype=jnp.int32) * V)[None, :]).reshape(-1)
    idx3 = flat_idx.reshape(NW, NCH, CH)
    table = emb_tables.reshape(F * V, D)
    embs = _make_sc_gather()(table, idx3).reshape(B, F * D)

    outT = _tc_fused(
        numerical_input.T, embs,
        bw0.T, bb0.reshape(-1, 1), bw1.T, bb1.reshape(-1, 1),
        bw2.T, bb2.reshape(-1, 1),
        tw0[:D].T, jnp.pad(tw0[D:D + NPAIR], ((0, 1), (0, 0))).T,
        tb0.reshape(-1, 1), tw1.T, tb1.reshape(-1, 1), tw2.T,
        tb2.reshape(-1, 1), tw3.T, tb3.reshape(-1, 1), tw4.T,
        tb4.reshape(-1, 1))
    return outT.T
